# baseline (speedup 1.0000x reference)
import functools
import numpy as np
import jax
import jax.numpy as jnp
from jax import lax
from jax.experimental import pallas as pl
from jax.experimental.pallas import tpu as pltpu
from jax.experimental.pallas import tpu_sc as plsc

CODEBOOK = 8192
D_EMB = 256
E_DIM = 2048
DIM = 1024

NW = 32
CHUNK = 128
N_IDX = 65536
PER_W = N_IDX // NW
N_CHUNK = PER_W // CHUNK


@functools.cache
def _make_gather():
    mesh = plsc.VectorSubcoreMesh(core_axis_name="c", subcore_axis_name="s")

    @functools.partial(
        pl.kernel,
        out_type=jax.ShapeDtypeStruct((N_IDX, D_EMB), jnp.float32),
        mesh=mesh,
        scratch_types=[
            pltpu.VMEM((N_CHUNK, CHUNK), jnp.int32),
            pltpu.VMEM((CHUNK, D_EMB), jnp.float32),
            pltpu.VMEM((CHUNK, D_EMB), jnp.float32),
            pltpu.SemaphoreType.DMA,
            pltpu.SemaphoreType.DMA,
        ],
    )
    def gather_k(table_hbm, idx_hbm, out_hbm, idx_v, buf0, buf1, sem0, sem1):
        wid = lax.axis_index("s") * 2 + lax.axis_index("c")
        pltpu.sync_copy(idx_hbm.at[pl.ds(wid * N_CHUNK, N_CHUNK)], idx_v)
        bufs = (buf0, buf1)
        sems = (sem0, sem1)
        descs = [None, None]
        descs[0] = pltpu.async_copy(table_hbm.at[idx_v.at[0]], bufs[0], sems[0])
        for c in range(N_CHUNK):
            if c + 1 < N_CHUNK:
                descs[(c + 1) % 2] = pltpu.async_copy(
                    table_hbm.at[idx_v.at[c + 1]], bufs[(c + 1) % 2],
                    sems[(c + 1) % 2])
            descs[c % 2].wait()
            pltpu.sync_copy(
                bufs[c % 2],
                out_hbm.at[pl.ds(wid * PER_W + c * CHUNK, CHUNK)])

    return gather_k


_BM = 1024
_SL = 2048


def _mm_body(alpha_ref, emb_ref, w_ref, out_ref):
    i = pl.program_id(0)
    base = (i % (_SL // _BM)) * _BM
    r = lax.broadcasted_iota(jnp.int32, (_BM, E_DIM), 0).astype(jnp.float32)
    c = lax.broadcasted_iota(jnp.int32, (_BM, E_DIM), 1)
    p = (c >> 8).astype(jnp.float32)
    d = c & 255
    div = jnp.exp((d & ~1).astype(jnp.float32) * (-np.log(10000.0) / D_EMB))
    pos = (jnp.float32(base) + r) * 8.0 + p
    pe = jnp.sin(pos * div + (d & 1).astype(jnp.float32) * np.float32(np.pi / 2))
    z = emb_ref[...] + alpha_ref[0] * pe
    out_ref[...] = lax.dot_general(
        z.astype(jnp.bfloat16), w_ref[...], (((1,), (1,)), ((), ())),
        preferred_element_type=jnp.float32)


def _matmul(alpha, emb2, W_out):
    m = emb2.shape[0]
    grid = (m // _BM,)
    return pl.pallas_call(
        _mm_body,
        grid=grid,
        in_specs=[
            pl.BlockSpec(memory_space=pltpu.SMEM),
            pl.BlockSpec((_BM, E_DIM), lambda i: (i, 0)),
            pl.BlockSpec((DIM, E_DIM), lambda i: (0, 0)),
        ],
        out_specs=pl.BlockSpec((_BM, DIM), lambda i: (i, 0)),
        out_shape=jax.ShapeDtypeStruct((m, DIM), jnp.float32),
    )(alpha, emb2, W_out.astype(jnp.bfloat16))


def kernel(x, W_emb, alpha, W_out):
    bs, sl, P = x.shape
    idx = x.reshape(N_IDX // CHUNK, CHUNK)
    emb = _make_gather()(W_emb, idx)
    emb2 = emb.reshape(bs * sl, E_DIM)
    out = _matmul(alpha, emb2, W_out)
    return out.reshape(bs, sl, DIM)


# speedup vs baseline: 1.3065x; 1.3065x over previous
import functools
import numpy as np
import jax
import jax.numpy as jnp
from jax import lax
from jax.experimental import pallas as pl
from jax.experimental.pallas import tpu as pltpu
from jax.experimental.pallas import tpu_sc as plsc

CODEBOOK = 8192
D_EMB = 256
E_DIM = 2048
DIM = 1024

NW = 32
CHUNK = 128
N_IDX = 65536
PER_W = N_IDX // NW
N_CHUNK = PER_W // CHUNK


@functools.cache
def _make_gather():
    mesh = plsc.VectorSubcoreMesh(core_axis_name="c", subcore_axis_name="s")

    @functools.partial(
        pl.kernel,
        out_type=jax.ShapeDtypeStruct((N_IDX, D_EMB), jnp.float32),
        mesh=mesh,
        scratch_types=[
            pltpu.VMEM((N_CHUNK, CHUNK), jnp.int32),
            pltpu.VMEM((CHUNK, D_EMB), jnp.float32),
            pltpu.VMEM((CHUNK, D_EMB), jnp.float32),
            pltpu.SemaphoreType.DMA,
            pltpu.SemaphoreType.DMA,
        ],
    )
    def gather_k(table_hbm, idx_hbm, out_hbm, idx_v, buf0, buf1, sem0, sem1):
        wid = lax.axis_index("s") * 2 + lax.axis_index("c")
        pltpu.sync_copy(idx_hbm.at[pl.ds(wid * N_CHUNK, N_CHUNK)], idx_v)
        bufs = (buf0, buf1)
        sems = (sem0, sem1)
        descs = [None, None]
        descs[0] = pltpu.async_copy(table_hbm.at[idx_v.at[0]], bufs[0], sems[0])
        for c in range(N_CHUNK):
            if c + 1 < N_CHUNK:
                descs[(c + 1) % 2] = pltpu.async_copy(
                    table_hbm.at[idx_v.at[c + 1]], bufs[(c + 1) % 2],
                    sems[(c + 1) % 2])
            descs[c % 2].wait()
            pltpu.sync_copy(
                bufs[c % 2],
                out_hbm.at[pl.ds(wid * PER_W + c * CHUNK, CHUNK)])

    return gather_k


_BM = 512
_SL = 2048


def _mm_body(alpha_ref, emb_ref, pe_ref, w_ref, out_ref):
    z = emb_ref[0] + alpha_ref[0] * pe_ref[...]
    out_ref[0] = lax.dot_general(
        z.astype(jnp.bfloat16), w_ref[...], (((1,), (1,)), ((), ())),
        preferred_element_type=jnp.float32)


def _matmul(alpha, emb3, pe2, W_out):
    bs = emb3.shape[0]
    grid = (_SL // _BM, bs)
    return pl.pallas_call(
        _mm_body,
        grid=grid,
        in_specs=[
            pl.BlockSpec(memory_space=pltpu.SMEM),
            pl.BlockSpec((1, _BM, E_DIM), lambda i, j: (j, i, 0)),
            pl.BlockSpec((_BM, E_DIM), lambda i, j: (i, 0)),
            pl.BlockSpec((DIM, E_DIM), lambda i, j: (0, 0)),
        ],
        out_specs=pl.BlockSpec((1, _BM, DIM), lambda i, j: (j, i, 0)),
        out_shape=jax.ShapeDtypeStruct((bs, _SL, DIM), jnp.float32),
    )(alpha, emb3, pe2, W_out.astype(jnp.bfloat16))


def _pe_table():
    pos = jnp.arange(16384, dtype=jnp.float32)[:, None]
    div = jnp.exp(jnp.arange(0, D_EMB, 2, dtype=jnp.float32)
                  * (-np.log(10000.0) / D_EMB))
    pe = jnp.zeros((16384, D_EMB), dtype=jnp.float32)
    pe = pe.at[:, 0::2].set(jnp.sin(pos * div))
    pe = pe.at[:, 1::2].set(jnp.cos(pos * div))
    return pe.reshape(E_DIM, E_DIM)


def kernel(x, W_emb, alpha, W_out):
    bs, sl, P = x.shape
    idx = x.reshape(N_IDX // CHUNK, CHUNK)
    emb = _make_gather()(W_emb, idx)
    emb3 = emb.reshape(bs, sl, E_DIM)
    out = _matmul(alpha, emb3, _pe_table(), W_out)
    return out.reshape(bs, sl, DIM)
